# baseline (speedup 1.0000x reference)
import functools

import jax
import jax.numpy as jnp
from jax import lax
from jax.experimental import pallas as pl
from jax.experimental.pallas import tpu as pltpu
from jax.experimental.pallas import tpu_sc as plsc

N, E, D, H = 10000, 160000, 256, 1024
DH = D // 2
NS = 16
CH = 128
KI = 80
EPS = KI * CH
EPAD = EPS * NS
NA = N + 16
ZR = 632
NB = 2
KH = KI // 2
BLK = 1000


def _mlp_body(h_ref, norm_ref, w0, b0, w1, b1, w2, b2, o0_ref, o1_ref):
    hn = h_ref[...] * norm_ref[...]
    f32 = jnp.float32
    x = jnp.maximum(jnp.dot(hn, w0[...], preferred_element_type=f32) + b0[...], 0.0)
    x = jnp.maximum(jnp.dot(x, w1[...], preferred_element_type=f32) + b1[...], 0.0)
    x = jnp.dot(x, w2[...], preferred_element_type=f32) + b2[...]
    o0_ref[...] = x[:, :DH]
    o1_ref[...] = x[:, DH:]


def _mlp(h, norm, w0, b0, w1, b1, w2, b2):
    grid = (N // BLK,)
    row_spec = lambda c: pl.BlockSpec((BLK, c), lambda i: (i, 0))
    w_spec = lambda r, c: pl.BlockSpec((r, c), lambda i: (0, 0))
    return pl.pallas_call(
        _mlp_body,
        grid=grid,
        in_specs=[
            row_spec(D), row_spec(1),
            w_spec(D, H), w_spec(1, H), w_spec(H, H), w_spec(1, H), w_spec(H, D), w_spec(1, D),
        ],
        out_specs=[row_spec(DH), row_spec(DH)],
        out_shape=[
            jax.ShapeDtypeStruct((N, DH), jnp.float32),
            jax.ShapeDtypeStruct((N, DH), jnp.float32),
        ],
    )(h, norm, w0, b0, w1, b1, w2, b2)


def _seg_sum(bh0, bh1, srcm, dstm, zer):
    mesh = plsc.VectorSubcoreMesh(core_axis_name="c", subcore_axis_name="s")

    @functools.partial(
        pl.kernel,
        out_type=[
            jax.ShapeDtypeStruct((N, DH), jnp.float32),
            jax.ShapeDtypeStruct((N, DH), jnp.float32),
        ],
        mesh=mesh,
        scratch_types=[
            pltpu.VMEM_SHARED((NA, DH), jnp.float32),
            pltpu.VMEM((KH, CH), jnp.int32),
            pltpu.VMEM((KH, CH), jnp.int32),
            pltpu.VMEM((NB, CH, DH), jnp.float32),
            [pltpu.SemaphoreType.DMA] * NB,
            [pltpu.SemaphoreType.DMA] * NB,
        ],
    )
    def seg(bh0_h, bh1_h, srcm_h, dstm_h, zer_h, out0_h, out1_h,
            acc, src_v, dst_v, rows_v, gsem, ssem):
        cid = lax.axis_index("c")
        sid = lax.axis_index("s")

        def body(bh_h, out_h):
            zbase = pl.multiple_of(jnp.minimum(sid * ZR, NA - ZR), 8)
            obase = pl.multiple_of(jnp.minimum(sid * ZR, N - ZR), 8)
            pltpu.sync_copy(zer_h, acc.at[pl.ds(zbase, ZR)])

            def gather(k, b):
                pltpu.async_copy(bh_h.at[src_v.at[k]], rows_v.at[b], gsem[b])

            def gather_wait(k, b):
                pltpu.make_async_copy(bh_h.at[src_v.at[k]], rows_v.at[b],
                                      gsem[b]).wait()

            def scatter(k, b):
                pltpu.async_copy(rows_v.at[b], acc.at[dst_v.at[k]], ssem[b],
                                 add=True)

            def scatter_wait(k, b):
                pltpu.make_async_copy(rows_v.at[b], acc.at[dst_v.at[k]],
                                      ssem[b]).wait()

            first = True
            for half in range(KI // KH):
                base = sid * KI + half * KH
                pltpu.sync_copy(srcm_h.at[pl.ds(base, KH)], src_v)
                pltpu.sync_copy(dstm_h.at[pl.ds(base, KH)], dst_v)
                if first:
                    plsc.subcore_barrier()
                    first = False
                for b in range(NB):
                    gather(b, b)

                def group(g, carry):
                    for b in range(NB):
                        k = g * NB + b
                        gather_wait(k, b)
                        scatter(k, b)
                    for b in range(NB):
                        kn = g * NB + NB + b
                        scatter_wait(kn - NB, b)
                        gather(kn, b)
                    return carry

                lax.fori_loop(0, KH // NB - 1, group, 0)
                for b in range(NB):
                    k = KH - NB + b
                    gather_wait(k, b)
                    scatter(k, b)
                for b in range(NB):
                    scatter_wait(KH - NB + b, b)
            plsc.subcore_barrier()
            pltpu.sync_copy(acc.at[pl.ds(obase, ZR)],
                            out_h.at[pl.ds(obase, ZR)])

        pl.when(cid == 0)(lambda: body(bh0_h, out0_h))
        pl.when(cid == 1)(lambda: body(bh1_h, out1_h))

    return seg(bh0, bh1, srcm, dstm, zer)


def _combine_body(ah0_ref, ah1_ref, a0_ref, a1_ref, norm_ref, out_ref):
    nrm = norm_ref[...]
    out_ref[:, :DH] = (ah0_ref[...] + a0_ref[...]) * nrm
    out_ref[:, DH:] = (ah1_ref[...] + a1_ref[...]) * nrm


def _combine(ah0, ah1, a0, a1, norm):
    grid = (N // BLK,)
    row_spec = lambda c: pl.BlockSpec((BLK, c), lambda i: (i, 0))
    return pl.pallas_call(
        _combine_body,
        grid=grid,
        in_specs=[row_spec(DH), row_spec(DH), row_spec(DH), row_spec(DH), row_spec(1)],
        out_specs=row_spec(D),
        out_shape=jax.ShapeDtypeStruct((N, D), jnp.float32),
    )(ah0, ah1, a0, a1, norm)


def kernel(h, e, norm, edge_index,
           A_W0, A_b0, A_W1, A_b1, A_W2, A_b2,
           B_W0, B_b0, B_W1, B_b1, B_W2, B_b2):
    src = edge_index[0].astype(jnp.int32)
    dst = edge_index[1].astype(jnp.int32)
    pad = EPAD - E
    srcm = jnp.concatenate([src, jnp.zeros((pad,), jnp.int32)]).reshape(-1, CH)
    dstm = jnp.concatenate([dst, jnp.full((pad,), N, jnp.int32)]).reshape(-1, CH)
    zer = jnp.zeros((ZR, DH), jnp.float32)

    bh0, bh1 = _mlp(h, norm, B_W0, B_b0.reshape(1, H), B_W1, B_b1.reshape(1, H),
                    B_W2, B_b2.reshape(1, D))
    agg0, agg1 = _seg_sum(bh0, bh1, srcm, dstm, zer)
    ah0, ah1 = _mlp(h, norm, A_W0, A_b0.reshape(1, H), A_W1, A_b1.reshape(1, H),
                    A_W2, A_b2.reshape(1, D))
    h_new = _combine(ah0, ah1, agg0, agg1, norm)
    return (h_new, e)


# speedup vs baseline: 1.1299x; 1.1299x over previous
import functools

import jax
import jax.numpy as jnp
from jax import lax
from jax.experimental import pallas as pl
from jax.experimental.pallas import tpu as pltpu
from jax.experimental.pallas import tpu_sc as plsc

N, E, D, H = 10000, 160000, 256, 1024
DH = D // 2
DQ = DH // 2
NS = 16
CH = 64
KI = 160
EPS = KI * CH
EPAD = EPS * NS
NA = N + 16
ZR = 632
NB = 4
KH = 40
BLK = 1000


def _mlp_body(h_ref, norm_ref, w0, b0, w1, b1, w2, b2, o0_ref, o1_ref):
    hn = h_ref[...] * norm_ref[...]
    f32 = jnp.float32
    x = jnp.maximum(jnp.dot(hn, w0[...], preferred_element_type=f32) + b0[...], 0.0)
    x = jnp.maximum(jnp.dot(x, w1[...], preferred_element_type=f32) + b1[...], 0.0)
    x = jnp.dot(x, w2[...], preferred_element_type=f32) + b2[...]
    o0_ref[...] = x[:, :DH]
    o1_ref[...] = x[:, DH:]


def _mlp(h, norm, w0, b0, w1, b1, w2, b2):
    grid = (N // BLK,)
    row_spec = lambda c: pl.BlockSpec((BLK, c), lambda i: (i, 0))
    w_spec = lambda r, c: pl.BlockSpec((r, c), lambda i: (0, 0))
    return pl.pallas_call(
        _mlp_body,
        grid=grid,
        in_specs=[
            row_spec(D), row_spec(1),
            w_spec(D, H), w_spec(1, H), w_spec(H, H), w_spec(1, H), w_spec(H, D), w_spec(1, D),
        ],
        out_specs=[row_spec(DH), row_spec(DH)],
        out_shape=[
            jax.ShapeDtypeStruct((N, DH), jnp.float32),
            jax.ShapeDtypeStruct((N, DH), jnp.float32),
        ],
    )(h, norm, w0, b0, w1, b1, w2, b2)


def _pack_pair(x):
    lo = lax.bitcast_convert_type(x[:, :DQ].astype(jnp.bfloat16), jnp.uint16)
    hi = lax.bitcast_convert_type(x[:, DQ:].astype(jnp.bfloat16), jnp.uint16)
    return lo.astype(jnp.int32) | (hi.astype(jnp.int32) << 16)


def _mlpb_body(h_ref, norm_ref, w0, b0, w1, b1, w2, b2, o0_ref, o1_ref):
    hn = h_ref[...] * norm_ref[...]
    f32 = jnp.float32
    x = jnp.maximum(jnp.dot(hn, w0[...], preferred_element_type=f32) + b0[...], 0.0)
    x = jnp.maximum(jnp.dot(x, w1[...], preferred_element_type=f32) + b1[...], 0.0)
    x = jnp.dot(x, w2[...], preferred_element_type=f32) + b2[...]
    o0_ref[...] = _pack_pair(x[:, :DH])
    o1_ref[...] = _pack_pair(x[:, DH:])


def _mlpb(h, norm, w0, b0, w1, b1, w2, b2):
    grid = (N // BLK,)
    row_spec = lambda c: pl.BlockSpec((BLK, c), lambda i: (i, 0))
    w_spec = lambda r, c: pl.BlockSpec((r, c), lambda i: (0, 0))
    return pl.pallas_call(
        _mlpb_body,
        grid=grid,
        in_specs=[
            row_spec(D), row_spec(1),
            w_spec(D, H), w_spec(1, H), w_spec(H, H), w_spec(1, H), w_spec(H, D), w_spec(1, D),
        ],
        out_specs=[row_spec(DQ), row_spec(DQ)],
        out_shape=[
            jax.ShapeDtypeStruct((N, DQ), jnp.int32),
            jax.ShapeDtypeStruct((N, DQ), jnp.int32),
        ],
    )(h, norm, w0, b0, w1, b1, w2, b2)


def _seg_sum(bp0, bp1, srcm, dstm, zer):
    mesh = plsc.VectorSubcoreMesh(core_axis_name="c", subcore_axis_name="s")

    @functools.partial(
        pl.kernel,
        out_type=[
            jax.ShapeDtypeStruct((N, DH), jnp.float32),
            jax.ShapeDtypeStruct((N, DH), jnp.float32),
        ],
        mesh=mesh,
        scratch_types=[
            pltpu.VMEM_SHARED((NA, DH), jnp.float32),
            pltpu.VMEM((KH, CH), jnp.int32),
            pltpu.VMEM((KH, CH), jnp.int32),
            pltpu.VMEM((NB, CH, DH), jnp.float32),
            [pltpu.SemaphoreType.DMA] * NB,
            [pltpu.SemaphoreType.DMA] * NB,
        ],
    )
    def seg(bp0_h, bp1_h, srcm_h, dstm_h, zer_h, out0_h, out1_h,
            acc, src_v, dst_v, rows_v, gsem, ssem):
        cid = lax.axis_index("c")
        sid = lax.axis_index("s")

        def body(bp_h, out_h):
            zbase = pl.multiple_of(jnp.minimum(sid * ZR, NA - ZR), 8)
            obase = pl.multiple_of(jnp.minimum(sid * ZR, N - ZR), 8)
            pltpu.sync_copy(zer_h, acc.at[pl.ds(zbase, ZR)])

            def gather(k, b):
                pltpu.async_copy(bp_h.at[src_v.at[k]], rows_v.at[b], gsem[b])

            def gather_wait(k, b):
                pltpu.make_async_copy(bp_h.at[src_v.at[k]], rows_v.at[b],
                                      gsem[b]).wait()

            def scatter(k, b):
                pltpu.async_copy(rows_v.at[b], acc.at[dst_v.at[k]], ssem[b],
                                 add=True)

            def scatter_wait(k, b):
                pltpu.make_async_copy(rows_v.at[b], acc.at[dst_v.at[k]],
                                      ssem[b]).wait()

            def step(k, b, do_swait, do_gather):
                gather_wait(k, b)
                if do_swait:
                    scatter_wait(k - NB, b)
                scatter(k, b)
                if do_gather:
                    gather(k + NB, b)

            first = True
            for half in range(KI // KH):
                base = sid * KI + half * KH
                pltpu.sync_copy(srcm_h.at[pl.ds(base, KH)], src_v)
                pltpu.sync_copy(dstm_h.at[pl.ds(base, KH)], dst_v)
                if first:
                    plsc.subcore_barrier()
                    first = False
                for b in range(NB):
                    gather(b, b)
                for b in range(NB):
                    step(b, b, False, True)

                def group(g, carry):
                    for b in range(NB):
                        step(g * NB + b, b, True, True)
                    return carry

                lax.fori_loop(1, KH // NB - 1, group, 0)
                for b in range(NB):
                    step(KH - NB + b, b, True, False)
                for b in range(NB):
                    scatter_wait(KH - NB + b, b)
            plsc.subcore_barrier()
            pltpu.sync_copy(acc.at[pl.ds(obase, ZR)],
                            out_h.at[pl.ds(obase, ZR)])

        pl.when(cid == 0)(lambda: body(bp0_h, out0_h))
        pl.when(cid == 1)(lambda: body(bp1_h, out1_h))

    return seg(bp0, bp1, srcm, dstm, zer)


def _combine_body(ah0_ref, ah1_ref, a0_ref, a1_ref, norm_ref, out_ref):
    nrm = norm_ref[...]
    out_ref[:, :DH] = (ah0_ref[...] + a0_ref[...]) * nrm
    out_ref[:, DH:] = (ah1_ref[...] + a1_ref[...]) * nrm


def _combine(ah0, ah1, a0, a1, norm):
    grid = (N // BLK,)
    row_spec = lambda c: pl.BlockSpec((BLK, c), lambda i: (i, 0))
    return pl.pallas_call(
        _combine_body,
        grid=grid,
        in_specs=[row_spec(DH), row_spec(DH), row_spec(DH), row_spec(DH), row_spec(1)],
        out_specs=row_spec(D),
        out_shape=jax.ShapeDtypeStruct((N, D), jnp.float32),
    )(ah0, ah1, a0, a1, norm)


def kernel(h, e, norm, edge_index,
           A_W0, A_b0, A_W1, A_b1, A_W2, A_b2,
           B_W0, B_b0, B_W1, B_b1, B_W2, B_b2):
    src = edge_index[0].astype(jnp.int32)
    dst = edge_index[1].astype(jnp.int32)
    pad = EPAD - E
    srcm = jnp.concatenate([src, jnp.zeros((pad,), jnp.int32)]).reshape(-1, CH)
    dstm = jnp.concatenate([dst, jnp.full((pad,), N, jnp.int32)]).reshape(-1, CH)
    zer = jnp.zeros((ZR, DH), jnp.float32)

    bp0, bp1 = _mlp(h, norm, B_W0, B_b0.reshape(1, H), B_W1, B_b1.reshape(1, H),
                    B_W2, B_b2.reshape(1, D))
    agg0, agg1 = _seg_sum(bp0, bp1, srcm, dstm, zer)
    ah0, ah1 = _mlp(h, norm, A_W0, A_b0.reshape(1, H), A_W1, A_b1.reshape(1, H),
                    A_W2, A_b2.reshape(1, D))
    h_new = _combine(ah0, ah1, agg0, agg1, norm)
    return (h_new, e)
